# baseline (speedup 1.0000x reference)
import functools

import jax
import jax.numpy as jnp
from jax import lax
from jax.experimental import pallas as pl
from jax.experimental.pallas import tpu as pltpu
from jax.experimental.pallas import tpu_sc as plsc

N = 10000
E = 320000
D_IN = 128
HID = 64

NC = 2
NS = 16
NW = NC * NS
CH = 128
NBUF = 4
CPT = -(-E // (NW * CH * NBUF)) * NBUF
E_PAD = NW * CPT * CH
N_PAD = 10240
RPT = N_PAD // NS
RB = 128
R = 1024
GRID = N_PAD // R


def _make_sc_agg(with_deg: bool, width: int):
    mesh = plsc.VectorSubcoreMesh(core_axis_name="c", subcore_axis_name="s",
                                  num_cores=NC, num_subcores=NS)
    out_type = [jax.ShapeDtypeStruct((NC, N_PAD, width), jnp.float32)]
    if with_deg:
        out_type.append(jax.ShapeDtypeStruct((NC, N_PAD), jnp.float32))

    scratch = [
        pltpu.VMEM((CPT, CH), jnp.int32),
        pltpu.VMEM((CPT, CH), jnp.int32),
        pltpu.VMEM((2, CH, width), jnp.float32),
        pltpu.VMEM_SHARED((N_PAD, width), jnp.float32),
        pltpu.SemaphoreType.DMA((2,)),
        pltpu.SemaphoreType.DMA((2,)),
    ]
    if with_deg:
        scratch += [
            pltpu.VMEM((CH,), jnp.float32),
            pltpu.VMEM((RPT,), jnp.float32),
            pltpu.VMEM_SHARED((N_PAD,), jnp.float32),
        ]

    def body(y_hbm, src_hbm, dst_hbm, out_hbm, *rest):
        if with_deg:
            (deg_hbm, src_v, dst_v, rows_v, acc, gsem, ssem,
             ones_v, dz_v, dega) = rest
        else:
            (src_v, dst_v, rows_v, acc, gsem, ssem) = rest

        c = lax.axis_index("c")
        s = lax.axis_index("s")
        wid = s * NC + c

        zero16 = jnp.zeros((16,), jnp.float32)

        def zrow(i, _):
            for j in range(width // 16):
                rows_v[0, i, pl.ds(j * 16, 16)] = zero16
            return 0

        lax.fori_loop(0, CH, zrow, 0)
        for t in range(RPT // RB):
            pltpu.sync_copy(rows_v.at[0], acc.at[pl.ds(s * RPT + t * RB, RB)])

        if with_deg:
            for j in range(CH // 16):
                ones_v[pl.ds(j * 16, 16)] = jnp.ones((16,), jnp.float32)
            for j in range(RPT // 16):
                dz_v[pl.ds(j * 16, 16)] = zero16
            pltpu.sync_copy(dz_v, dega.at[pl.ds(s * RPT, RPT)])

        pltpu.sync_copy(src_hbm.at[wid], src_v)
        pltpu.sync_copy(dst_hbm.at[wid], dst_v)

        plsc.subcore_barrier()

        pltpu.async_copy(y_hbm.at[src_v.at[0]], rows_v.at[0], gsem.at[0])

        def outer(o, _):
            for b in range(2):
                jj = o * 2 + b
                pltpu.make_async_copy(y_hbm.at[src_v.at[jj]],
                                      rows_v.at[b], gsem.at[b]).wait()

                @pl.when(jj + 1 < CPT)
                def _():
                    pltpu.async_copy(y_hbm.at[src_v.at[jj + 1]],
                                     rows_v.at[1 - b], gsem.at[1 - b])

                pltpu.sync_copy(rows_v.at[b], acc.at[dst_v.at[jj]],
                                add=True)
                if with_deg:
                    pltpu.sync_copy(ones_v, dega.at[dst_v.at[jj]], add=True)
            return 0

        lax.fori_loop(0, CPT // 2, outer, 0)

        plsc.subcore_barrier()

        for t in range(RPT // RB):
            r0 = s * RPT + t * RB
            pltpu.sync_copy(acc.at[pl.ds(r0, RB)], rows_v.at[0])
            pltpu.sync_copy(rows_v.at[0], out_hbm.at[c, pl.ds(r0, RB)])
        if with_deg:
            pltpu.sync_copy(dega.at[pl.ds(s * RPT, RPT)], dz_v)
            pltpu.sync_copy(dz_v, deg_hbm.at[c, pl.ds(s * RPT, RPT)])

    return pl.kernel(body, out_type=out_type, mesh=mesh,
                     scratch_types=scratch,
                     compiler_params=pltpu.CompilerParams(
                         use_tc_tiling_on_sc=False))


def _mm_body(x_ref, w_ref, o_ref):
    o_ref[...] = jnp.dot(x_ref[...], w_ref[...],
                         preferred_element_type=jnp.float32)


def _combine_body(p0, p1, d0, d1, yr, b, w, o_ref):
    rdeg = 1.0 / jnp.maximum(d0[...] + d1[...], 1.0)
    h = jnp.maximum((p0[...] + p1[...]) * rdeg + b[...] + yr[:, HID:], 0.0)
    o_ref[...] = jnp.dot(h, w[...], preferred_element_type=jnp.float32)


def _final_body(q0, q1, d0, d1, yr, b, wc, bc, o_ref):
    rdeg = 1.0 / jnp.maximum(d0[...] + d1[...], 1.0)
    h = jnp.maximum((q0[...] + q1[...]) * rdeg + b[...] + yr[:, HID:], 0.0)
    o_ref[...] = jnp.dot(h, wc[...], preferred_element_type=jnp.float32) + bc[...]


def _full(shape):
    nd = len(shape)
    return pl.BlockSpec(shape, lambda i: (0,) * nd)


_row64 = pl.BlockSpec((R, HID), lambda i: (i, 0))
_row128 = pl.BlockSpec((R, 2 * HID), lambda i: (i, 0))
_row1 = pl.BlockSpec((R, 1), lambda i: (i, 0))


def _mm(x, w):
    return pl.pallas_call(
        _mm_body,
        grid=(GRID,),
        in_specs=[pl.BlockSpec((R, x.shape[1]), lambda i: (i, 0)),
                  _full(w.shape)],
        out_specs=pl.BlockSpec((R, w.shape[1]), lambda i: (i, 0)),
        out_shape=jax.ShapeDtypeStruct((N_PAD, w.shape[1]), jnp.float32),
    )(x, w)


def _combine(p, d, yr, b, w):
    return pl.pallas_call(
        _combine_body,
        grid=(GRID,),
        in_specs=[_row64, _row64, _row1, _row1, _row128,
                  _full((1, HID)), _full((HID, 2 * HID))],
        out_specs=_row128,
        out_shape=jax.ShapeDtypeStruct((N_PAD, 2 * HID), jnp.float32),
    )(p[0], p[1], d[0, :, None], d[1, :, None], yr, b, w)


def _final(q, d, yr, b, wc, bc):
    return pl.pallas_call(
        _final_body,
        grid=(GRID,),
        in_specs=[_row64, _row64, _row1, _row1, _row128,
                  _full((1, HID)), _full((HID, 1)), _full((1, 1))],
        out_specs=_row1,
        out_shape=jax.ShapeDtypeStruct((N_PAD, 1), jnp.float32),
    )(q[0], q[1], d[0, :, None], d[1, :, None], yr, b, wc, bc)


_agg_cache = {}


def _get_agg(with_deg):
    if with_deg not in _agg_cache:
        _agg_cache[with_deg] = _make_sc_agg(with_deg, HID)
    return _agg_cache[with_deg]


@jax.jit
def kernel(x, edge_index, W1l, b1, W1r, W2l, b2, W2r, Wc, bc):
    ei = edge_index.astype(jnp.int32)
    pad = jnp.full((E_PAD - E,), N, jnp.int32)
    src = jnp.concatenate([ei[0], pad]).reshape(NW, CPT, CH)
    dst = jnp.concatenate([ei[1], pad]).reshape(NW, CPT, CH)

    x_pad = jnp.zeros((N_PAD, D_IN), jnp.float32).at[:N].set(x)
    wcat1 = jnp.concatenate([W1l.T, W1r.T], axis=1)
    wcat2 = jnp.concatenate([W2l.T, W2r.T], axis=1)

    yr1 = _mm(x_pad, wcat1)
    p1, deg = _get_agg(True)(yr1[:, :HID], src, dst)
    yr2 = _combine(p1, deg, yr1, b1.reshape(1, HID), wcat2)

    [p2] = _get_agg(False)(yr2[:, :HID], src, dst)
    logits = _final(p2, deg, yr2, b2.reshape(1, HID), Wc.T,
                    bc.reshape(1, 1))
    return logits[:N, 0]


# speedup vs baseline: 2.0294x; 2.0294x over previous
import functools

import jax
import jax.numpy as jnp
from jax import lax
from jax.experimental import pallas as pl
from jax.experimental.pallas import tpu as pltpu
from jax.experimental.pallas import tpu_sc as plsc

N = 10000
E = 320000
D_IN = 128
HID = 64

NC = 2
NS = 16
NW = NC * NS
CH = 128
NBUF = 4
CPT = -(-E // (NW * CH * NBUF)) * NBUF
E_PAD = NW * CPT * CH
N_PAD = 10240
RPT = N_PAD // NS
RB = 128
R = 1024
GRID = N_PAD // R


def _make_sc_agg(with_deg: bool, width: int):
    mesh = plsc.VectorSubcoreMesh(core_axis_name="c", subcore_axis_name="s",
                                  num_cores=NC, num_subcores=NS)
    out_type = [jax.ShapeDtypeStruct((NC, N_PAD, width), jnp.float32)]
    if with_deg:
        out_type.append(jax.ShapeDtypeStruct((NC, N_PAD), jnp.float32))

    scratch = [
        pltpu.VMEM((CPT, CH), jnp.int32),
        pltpu.VMEM((CPT, CH), jnp.int32),
        pltpu.VMEM((CH, width), jnp.float32),
        pltpu.VMEM_SHARED((N_PAD, width), jnp.float32),
        pltpu.VMEM_SHARED((N_PAD, width), jnp.float32),
        pltpu.SemaphoreType.DMA,
    ]
    if with_deg:
        scratch += [
            pltpu.VMEM((CH,), jnp.float32),
            pltpu.VMEM((RPT,), jnp.float32),
            pltpu.VMEM_SHARED((N_PAD,), jnp.float32),
        ]

    def body(y_hbm, src_hbm, dst_hbm, out_hbm, *rest):
        if with_deg:
            (deg_hbm, src_v, dst_v, rows_v, acc, y_s, sem,
             ones_v, dz_v, dega) = rest
        else:
            (src_v, dst_v, rows_v, acc, y_s, sem) = rest

        c = lax.axis_index("c")
        s = lax.axis_index("s")
        wid = s * NC + c

        zero16 = jnp.zeros((16,), jnp.float32)

        def zrow(i, _):
            for j in range(width // 16):
                rows_v[i, pl.ds(j * 16, 16)] = zero16
            return 0

        lax.fori_loop(0, CH, zrow, 0)
        for t in range(RPT // RB):
            r0 = s * RPT + t * RB
            pltpu.sync_copy(rows_v, acc.at[pl.ds(r0, RB)])

        if with_deg:
            for j in range(CH // 16):
                ones_v[pl.ds(j * 16, 16)] = jnp.ones((16,), jnp.float32)
            for j in range(RPT // 16):
                dz_v[pl.ds(j * 16, 16)] = zero16
            pltpu.sync_copy(dz_v, dega.at[pl.ds(s * RPT, RPT)])

        pltpu.sync_copy(src_hbm.at[wid], src_v)
        pltpu.sync_copy(dst_hbm.at[wid], dst_v)
        for t in range(RPT // RB):
            r0 = s * RPT + t * RB
            pltpu.sync_copy(y_hbm.at[pl.ds(r0, RB)], rows_v)
            pltpu.sync_copy(rows_v, y_s.at[pl.ds(r0, RB)])

        plsc.subcore_barrier()

        def step(j, _):
            pltpu.async_copy(y_s.at[src_v.at[j]], rows_v, sem).wait()
            pltpu.sync_copy(rows_v, acc.at[dst_v.at[j]], add=True)
            if with_deg:
                pltpu.sync_copy(ones_v, dega.at[dst_v.at[j]], add=True)
            return 0

        lax.fori_loop(0, CPT, step, 0)

        plsc.subcore_barrier()

        for t in range(RPT // RB):
            r0 = s * RPT + t * RB
            pltpu.sync_copy(acc.at[pl.ds(r0, RB)], rows_v)
            pltpu.sync_copy(rows_v, out_hbm.at[c, pl.ds(r0, RB)])
        if with_deg:
            pltpu.sync_copy(dega.at[pl.ds(s * RPT, RPT)], dz_v)
            pltpu.sync_copy(dz_v, deg_hbm.at[c, pl.ds(s * RPT, RPT)])

    return pl.kernel(body, out_type=out_type, mesh=mesh,
                     scratch_types=scratch,
                     compiler_params=pltpu.CompilerParams(
                         use_tc_tiling_on_sc=False))


def _mm_body(x_ref, w_ref, o_ref):
    o_ref[...] = jnp.dot(x_ref[...], w_ref[...],
                         preferred_element_type=jnp.float32)


def _combine_body(p0, p1, d0, d1, yr, b, w, o_ref):
    rdeg = 1.0 / jnp.maximum(d0[...] + d1[...], 1.0)
    h = jnp.maximum((p0[...] + p1[...]) * rdeg + b[...] + yr[:, HID:], 0.0)
    o_ref[...] = jnp.dot(h, w[...], preferred_element_type=jnp.float32)


def _final_body(q0, q1, d0, d1, yr, b, wc, bc, o_ref):
    rdeg = 1.0 / jnp.maximum(d0[...] + d1[...], 1.0)
    h = jnp.maximum((q0[...] + q1[...]) * rdeg + b[...] + yr[:, HID:], 0.0)
    o_ref[...] = jnp.dot(h, wc[...], preferred_element_type=jnp.float32) + bc[...]


def _full(shape):
    nd = len(shape)
    return pl.BlockSpec(shape, lambda i: (0,) * nd)


_row64 = pl.BlockSpec((R, HID), lambda i: (i, 0))
_row128 = pl.BlockSpec((R, 2 * HID), lambda i: (i, 0))
_row1 = pl.BlockSpec((R, 1), lambda i: (i, 0))


def _mm(x, w):
    return pl.pallas_call(
        _mm_body,
        grid=(GRID,),
        in_specs=[pl.BlockSpec((R, x.shape[1]), lambda i: (i, 0)),
                  _full(w.shape)],
        out_specs=pl.BlockSpec((R, w.shape[1]), lambda i: (i, 0)),
        out_shape=jax.ShapeDtypeStruct((N_PAD, w.shape[1]), jnp.float32),
    )(x, w)


def _combine(p, d, yr, b, w):
    return pl.pallas_call(
        _combine_body,
        grid=(GRID,),
        in_specs=[_row64, _row64, _row1, _row1, _row128,
                  _full((1, HID)), _full((HID, 2 * HID))],
        out_specs=_row128,
        out_shape=jax.ShapeDtypeStruct((N_PAD, 2 * HID), jnp.float32),
    )(p[0], p[1], d[0, :, None], d[1, :, None], yr, b, w)


def _final(q, d, yr, b, wc, bc):
    return pl.pallas_call(
        _final_body,
        grid=(GRID,),
        in_specs=[_row64, _row64, _row1, _row1, _row128,
                  _full((1, HID)), _full((HID, 1)), _full((1, 1))],
        out_specs=_row1,
        out_shape=jax.ShapeDtypeStruct((N_PAD, 1), jnp.float32),
    )(q[0], q[1], d[0, :, None], d[1, :, None], yr, b, wc, bc)


_agg_cache = {}


def _get_agg(with_deg):
    if with_deg not in _agg_cache:
        _agg_cache[with_deg] = _make_sc_agg(with_deg, HID)
    return _agg_cache[with_deg]


@jax.jit
def kernel(x, edge_index, W1l, b1, W1r, W2l, b2, W2r, Wc, bc):
    ei = edge_index.astype(jnp.int32)
    pad = jnp.full((E_PAD - E,), N, jnp.int32)
    src = jnp.concatenate([ei[0], pad]).reshape(NW, CPT, CH)
    dst = jnp.concatenate([ei[1], pad]).reshape(NW, CPT, CH)

    x_pad = jnp.zeros((N_PAD, D_IN), jnp.float32).at[:N].set(x)
    wcat1 = jnp.concatenate([W1l.T, W1r.T], axis=1)
    wcat2 = jnp.concatenate([W2l.T, W2r.T], axis=1)

    yr1 = _mm(x_pad, wcat1)
    p1, deg = _get_agg(True)(yr1[:, :HID], src, dst)
    yr2 = _combine(p1, deg, yr1, b1.reshape(1, HID), wcat2)

    [p2] = _get_agg(False)(yr2[:, :HID], src, dst)
    logits = _final(p2, deg, yr2, b2.reshape(1, HID), Wc.T,
                    bc.reshape(1, 1))
    return logits[:N, 0]


# speedup vs baseline: 2.0635x; 1.0168x over previous
import functools

import jax
import jax.numpy as jnp
from jax import lax
from jax.experimental import pallas as pl
from jax.experimental.pallas import tpu as pltpu
from jax.experimental.pallas import tpu_sc as plsc

N = 10000
E = 320000
D_IN = 128
HID = 64

NC = 2
NS = 16
NW = NC * NS
CH = 256
NBUF = 4
CPT = -(-E // (NW * CH * NBUF)) * NBUF
E_PAD = NW * CPT * CH
N_PAD = 10240
RPT = N_PAD // NS
RB = 128
R = 1024
GRID = N_PAD // R


def _make_sc_agg(with_deg: bool, width: int):
    mesh = plsc.VectorSubcoreMesh(core_axis_name="c", subcore_axis_name="s",
                                  num_cores=NC, num_subcores=NS)
    out_type = [jax.ShapeDtypeStruct((NC, N_PAD, width), jnp.float32)]
    if with_deg:
        out_type.append(jax.ShapeDtypeStruct((NC, N_PAD), jnp.float32))

    scratch = [
        pltpu.VMEM((CPT, CH), jnp.int32),
        pltpu.VMEM((CPT, CH), jnp.int32),
        pltpu.VMEM((CH, width), jnp.float32),
        pltpu.VMEM_SHARED((N_PAD, width), jnp.float32),
        pltpu.VMEM_SHARED((N_PAD, width), jnp.float32),
        pltpu.SemaphoreType.DMA,
    ]
    if with_deg:
        scratch += [
            pltpu.VMEM((CH,), jnp.float32),
            pltpu.VMEM((RPT,), jnp.float32),
            pltpu.VMEM_SHARED((N_PAD,), jnp.float32),
        ]

    def body(y_hbm, src_hbm, dst_hbm, out_hbm, *rest):
        if with_deg:
            (deg_hbm, src_v, dst_v, rows_v, acc, y_s, sem,
             ones_v, dz_v, dega) = rest
        else:
            (src_v, dst_v, rows_v, acc, y_s, sem) = rest

        c = lax.axis_index("c")
        s = lax.axis_index("s")
        wid = s * NC + c

        zero16 = jnp.zeros((16,), jnp.float32)

        def zrow(i, _):
            for j in range(width // 16):
                rows_v[i, pl.ds(j * 16, 16)] = zero16
            return 0

        lax.fori_loop(0, CH, zrow, 0)
        rv_rb = rows_v.at[pl.ds(0, RB)]
        for t in range(RPT // RB):
            r0 = s * RPT + t * RB
            pltpu.sync_copy(rv_rb, acc.at[pl.ds(r0, RB)])

        if with_deg:
            for j in range(CH // 16):
                ones_v[pl.ds(j * 16, 16)] = jnp.ones((16,), jnp.float32)
            for j in range(RPT // 16):
                dz_v[pl.ds(j * 16, 16)] = zero16
            pltpu.sync_copy(dz_v, dega.at[pl.ds(s * RPT, RPT)])

        pltpu.sync_copy(src_hbm.at[wid], src_v)
        pltpu.sync_copy(dst_hbm.at[wid], dst_v)
        for t in range(RPT // RB):
            r0 = s * RPT + t * RB
            pltpu.sync_copy(y_hbm.at[pl.ds(r0, RB)], rv_rb)
            pltpu.sync_copy(rv_rb, y_s.at[pl.ds(r0, RB)])

        plsc.subcore_barrier()

        def step(j, _):
            pltpu.async_copy(y_s.at[src_v.at[j]], rows_v, sem).wait()
            pltpu.sync_copy(rows_v, acc.at[dst_v.at[j]], add=True)
            if with_deg:
                pltpu.sync_copy(ones_v, dega.at[dst_v.at[j]], add=True)
            return 0

        lax.fori_loop(0, CPT, step, 0)

        plsc.subcore_barrier()

        for t in range(RPT // RB):
            r0 = s * RPT + t * RB
            pltpu.sync_copy(acc.at[pl.ds(r0, RB)], rv_rb)
            pltpu.sync_copy(rv_rb, out_hbm.at[c, pl.ds(r0, RB)])
        if with_deg:
            pltpu.sync_copy(dega.at[pl.ds(s * RPT, RPT)], dz_v)
            pltpu.sync_copy(dz_v, deg_hbm.at[c, pl.ds(s * RPT, RPT)])

    return pl.kernel(body, out_type=out_type, mesh=mesh,
                     scratch_types=scratch,
                     compiler_params=pltpu.CompilerParams(
                         use_tc_tiling_on_sc=False))


def _mm_body(x_ref, w_ref, o_ref):
    o_ref[...] = jnp.dot(x_ref[...], w_ref[...],
                         preferred_element_type=jnp.float32)


def _combine_body(p0, p1, d0, d1, yr, b, w, o_ref):
    rdeg = 1.0 / jnp.maximum(d0[...] + d1[...], 1.0)
    h = jnp.maximum((p0[...] + p1[...]) * rdeg + b[...] + yr[:, HID:], 0.0)
    o_ref[...] = jnp.dot(h, w[...], preferred_element_type=jnp.float32)


def _final_body(q0, q1, d0, d1, yr, b, wc, bc, o_ref):
    rdeg = 1.0 / jnp.maximum(d0[...] + d1[...], 1.0)
    h = jnp.maximum((q0[...] + q1[...]) * rdeg + b[...] + yr[:, HID:], 0.0)
    o_ref[...] = jnp.dot(h, wc[...], preferred_element_type=jnp.float32) + bc[...]


def _full(shape):
    nd = len(shape)
    return pl.BlockSpec(shape, lambda i: (0,) * nd)


_row64 = pl.BlockSpec((R, HID), lambda i: (i, 0))
_row128 = pl.BlockSpec((R, 2 * HID), lambda i: (i, 0))
_row1 = pl.BlockSpec((R, 1), lambda i: (i, 0))


def _mm(x, w):
    return pl.pallas_call(
        _mm_body,
        grid=(GRID,),
        in_specs=[pl.BlockSpec((R, x.shape[1]), lambda i: (i, 0)),
                  _full(w.shape)],
        out_specs=pl.BlockSpec((R, w.shape[1]), lambda i: (i, 0)),
        out_shape=jax.ShapeDtypeStruct((N_PAD, w.shape[1]), jnp.float32),
    )(x, w)


def _combine(p, d, yr, b, w):
    return pl.pallas_call(
        _combine_body,
        grid=(GRID,),
        in_specs=[_row64, _row64, _row1, _row1, _row128,
                  _full((1, HID)), _full((HID, 2 * HID))],
        out_specs=_row128,
        out_shape=jax.ShapeDtypeStruct((N_PAD, 2 * HID), jnp.float32),
    )(p[0], p[1], d[0, :, None], d[1, :, None], yr, b, w)


def _final(q, d, yr, b, wc, bc):
    return pl.pallas_call(
        _final_body,
        grid=(GRID,),
        in_specs=[_row64, _row64, _row1, _row1, _row128,
                  _full((1, HID)), _full((HID, 1)), _full((1, 1))],
        out_specs=_row1,
        out_shape=jax.ShapeDtypeStruct((N_PAD, 1), jnp.float32),
    )(q[0], q[1], d[0, :, None], d[1, :, None], yr, b, wc, bc)


_agg_cache = {}


def _get_agg(with_deg):
    if with_deg not in _agg_cache:
        _agg_cache[with_deg] = _make_sc_agg(with_deg, HID)
    return _agg_cache[with_deg]


@jax.jit
def kernel(x, edge_index, W1l, b1, W1r, W2l, b2, W2r, Wc, bc):
    ei = edge_index.astype(jnp.int32)
    pad = jnp.full((E_PAD - E,), N, jnp.int32)
    src = jnp.concatenate([ei[0], pad]).reshape(NW, CPT, CH)
    dst = jnp.concatenate([ei[1], pad]).reshape(NW, CPT, CH)

    x_pad = jnp.zeros((N_PAD, D_IN), jnp.float32).at[:N].set(x)
    wcat1 = jnp.concatenate([W1l.T, W1r.T], axis=1)
    wcat2 = jnp.concatenate([W2l.T, W2r.T], axis=1)

    yr1 = _mm(x_pad, wcat1)
    p1, deg = _get_agg(True)(yr1[:, :HID], src, dst)
    yr2 = _combine(p1, deg, yr1, b1.reshape(1, HID), wcat2)

    [p2] = _get_agg(False)(yr2[:, :HID], src, dst)
    logits = _final(p2, deg, yr2, b2.reshape(1, HID), Wc.T,
                    bc.reshape(1, 1))
    return logits[:N, 0]


# speedup vs baseline: 2.2043x; 1.0682x over previous
import jax
import jax.numpy as jnp
from jax import lax
from jax.experimental import pallas as pl
from jax.experimental.pallas import tpu as pltpu
from jax.experimental.pallas import tpu_sc as plsc

N = 10000
E = 320000
D_IN = 128
HID = 64

NC = 2
NS = 16
NW = NC * NS
CH = 256
CPT = -(-E // (NW * CH))
E_PAD = NW * CPT * CH
N_PAD = 10240
RPT = N_PAD // NS
RB = 128
R = 1024
GRID = N_PAD // R


def _make_sc_agg(with_deg: bool, width: int):
    mesh = plsc.VectorSubcoreMesh(core_axis_name="c", subcore_axis_name="s",
                                  num_cores=NC, num_subcores=NS)
    out_type = [jax.ShapeDtypeStruct((N_PAD, width), jnp.float32)] * 2
    if with_deg:
        out_type += [jax.ShapeDtypeStruct((N_PAD,), jnp.float32)] * 2

    scratch = [
        pltpu.VMEM((CPT, CH), jnp.int32),
        pltpu.VMEM((CPT, CH), jnp.int32),
        pltpu.VMEM((CH, width), jnp.float32),
        pltpu.VMEM_SHARED((N_PAD, width), jnp.float32),
        pltpu.VMEM_SHARED((N_PAD, width), jnp.float32),
        pltpu.SemaphoreType.DMA,
    ]
    if with_deg:
        scratch += [
            pltpu.VMEM((CH,), jnp.float32),
            pltpu.VMEM((RPT,), jnp.float32),
            pltpu.VMEM_SHARED((N_PAD,), jnp.float32),
        ]

    def body(y_hbm, src_hbm, dst_hbm, out0_hbm, out1_hbm, *rest):
        if with_deg:
            (deg0_hbm, deg1_hbm, src_v, dst_v, rows_v, acc, y_s, sem,
             ones_v, dz_v, dega) = rest
        else:
            (src_v, dst_v, rows_v, acc, y_s, sem) = rest

        c = lax.axis_index("c")
        s = lax.axis_index("s")
        wid = s * NC + c

        zero16 = jnp.zeros((16,), jnp.float32)

        def zrow(i, _):
            for j in range(width // 16):
                rows_v[i, pl.ds(j * 16, 16)] = zero16
            return 0

        lax.fori_loop(0, RB, zrow, 0)
        rv_rb = rows_v.at[pl.ds(0, RB)]
        for t in range(RPT // RB):
            r0 = s * RPT + t * RB
            pltpu.sync_copy(rv_rb, acc.at[pl.ds(r0, RB)])

        if with_deg:
            for j in range(CH // 16):
                ones_v[pl.ds(j * 16, 16)] = jnp.ones((16,), jnp.float32)
            for j in range(RPT // 16):
                dz_v[pl.ds(j * 16, 16)] = zero16
            pltpu.sync_copy(dz_v, dega.at[pl.ds(s * RPT, RPT)])

        pltpu.sync_copy(src_hbm.at[wid], src_v)
        pltpu.sync_copy(dst_hbm.at[wid], dst_v)
        for t in range(RPT // RB):
            r0 = s * RPT + t * RB
            pltpu.sync_copy(y_hbm.at[pl.ds(r0, RB)], rv_rb)
            pltpu.sync_copy(rv_rb, y_s.at[pl.ds(r0, RB)])

        plsc.subcore_barrier()

        def step(j, _):
            pltpu.async_copy(y_s.at[src_v.at[j]], rows_v, sem).wait()
            pltpu.sync_copy(rows_v, acc.at[dst_v.at[j]], add=True)
            if with_deg:
                pltpu.sync_copy(ones_v, dega.at[dst_v.at[j]], add=True)
            return 0

        lax.fori_loop(0, CPT, step, 0)

        plsc.subcore_barrier()

        def writeout(out_hbm, deg_hbm):
            for t in range(RPT // RB):
                r0 = s * RPT + t * RB
                pltpu.sync_copy(acc.at[pl.ds(r0, RB)], rv_rb)
                pltpu.sync_copy(rv_rb, out_hbm.at[pl.ds(r0, RB)])
            if with_deg:
                pltpu.sync_copy(dega.at[pl.ds(s * RPT, RPT)], dz_v)
                pltpu.sync_copy(dz_v, deg_hbm.at[pl.ds(s * RPT, RPT)])

        @pl.when(c == 0)
        def _():
            writeout(out0_hbm, rest[0] if with_deg else None)

        @pl.when(c == 1)
        def _():
            writeout(out1_hbm, rest[1] if with_deg else None)

    return pl.kernel(body, out_type=out_type, mesh=mesh,
                     scratch_types=scratch,
                     compiler_params=pltpu.CompilerParams(
                         use_tc_tiling_on_sc=False))


def _mm_body(x_ref, w_ref, y_ref, r_ref):
    y_ref[...] = jnp.dot(x_ref[...], w_ref[:, :HID],
                         preferred_element_type=jnp.float32)
    r_ref[...] = jnp.dot(x_ref[...], w_ref[:, HID:],
                         preferred_element_type=jnp.float32)


def _combine_body(p0, p1, d0, d1, r1, b, w, y_ref, r_ref):
    rdeg = 1.0 / jnp.maximum(d0[...] + d1[...], 1.0)
    h = jnp.maximum((p0[...] + p1[...]) * rdeg + b[...] + r1[...], 0.0)
    y_ref[...] = jnp.dot(h, w[:, :HID], preferred_element_type=jnp.float32)
    r_ref[...] = jnp.dot(h, w[:, HID:], preferred_element_type=jnp.float32)


def _final_body(q0, q1, d0, d1, r2, b, wc, bc, o_ref):
    rdeg = 1.0 / jnp.maximum(d0[...] + d1[...], 1.0)
    h = jnp.maximum((q0[...] + q1[...]) * rdeg + b[...] + r2[...], 0.0)
    o_ref[...] = jnp.sum(h * wc[...], axis=1) + bc[0, 0]


def _full(shape):
    nd = len(shape)
    return pl.BlockSpec(shape, lambda i: (0,) * nd)


_row64 = pl.BlockSpec((R, HID), lambda i: (i, 0))
_row1 = pl.BlockSpec((R, 1), lambda i: (i, 0))
_rowf = pl.BlockSpec((R,), lambda i: (i,))
_o64 = jax.ShapeDtypeStruct((N_PAD, HID), jnp.float32)


def _mm(x, w):
    return pl.pallas_call(
        _mm_body,
        grid=(GRID,),
        in_specs=[pl.BlockSpec((R, D_IN), lambda i: (i, 0)),
                  _full((D_IN, 2 * HID))],
        out_specs=[_row64, _row64],
        out_shape=[_o64, _o64],
    )(x, w)


def _combine(p0, p1, d0, d1, r1, b, w):
    return pl.pallas_call(
        _combine_body,
        grid=(GRID,),
        in_specs=[_row64, _row64, _row1, _row1, _row64,
                  _full((1, HID)), _full((HID, 2 * HID))],
        out_specs=[_row64, _row64],
        out_shape=[_o64, _o64],
    )(p0, p1, d0, d1, r1, b, w)


def _final(q0, q1, d0, d1, r2, b, wc, bc):
    return pl.pallas_call(
        _final_body,
        grid=(GRID,),
        in_specs=[_row64, _row64, _row1, _row1, _row64,
                  _full((1, HID)), _full((1, HID)), _full((1, 1))],
        out_specs=_rowf,
        out_shape=jax.ShapeDtypeStruct((N_PAD,), jnp.float32),
    )(q0, q1, d0, d1, r2, b, wc, bc)


_agg_cache = {}


def _get_agg(with_deg):
    if with_deg not in _agg_cache:
        _agg_cache[with_deg] = _make_sc_agg(with_deg, HID)
    return _agg_cache[with_deg]


@jax.jit
def kernel(x, edge_index, W1l, b1, W1r, W2l, b2, W2r, Wc, bc):
    ei = edge_index.astype(jnp.int32)
    pad = jnp.full((E_PAD - E,), N, jnp.int32)
    src = jnp.concatenate([ei[0], pad]).reshape(NW, CPT, CH)
    dst = jnp.concatenate([ei[1], pad]).reshape(NW, CPT, CH)

    x_pad = jnp.zeros((N_PAD, D_IN), jnp.float32).at[:N].set(x)
    wcat1 = jnp.concatenate([W1l.T, W1r.T], axis=1)
    wcat2 = jnp.concatenate([W2l.T, W2r.T], axis=1)

    y1, r1 = _mm(x_pad, wcat1)
    p0, p1, dg0, dg1 = _get_agg(True)(y1, src, dst)
    d0 = dg0.reshape(N_PAD, 1)
    d1 = dg1.reshape(N_PAD, 1)
    y2, r2 = _combine(p0, p1, d0, d1, r1, b1.reshape(1, HID), wcat2)

    q0, q1 = _get_agg(False)(y2, src, dst)
    logits = _final(q0, q1, d0, d1, r2, b2.reshape(1, HID),
                    Wc.reshape(1, HID), bc.reshape(1, 1))
    return logits[:N]


# speedup vs baseline: 2.2195x; 1.0069x over previous
import jax
import jax.numpy as jnp
from jax import lax
from jax.experimental import pallas as pl
from jax.experimental.pallas import tpu as pltpu
from jax.experimental.pallas import tpu_sc as plsc

N = 10000
E = 320000
D_IN = 128
HID = 64

NC = 2
NS = 16
NW = NC * NS
CH = 256
CPT = -(-E // (NW * CH))
E_PAD = NW * CPT * CH
N_PAD = 10240
RPT = N_PAD // NS
RB = 128
R = 1024
GRID = N_PAD // R


def _make_sc_agg(with_deg: bool, width: int):
    mesh = plsc.VectorSubcoreMesh(core_axis_name="c", subcore_axis_name="s",
                                  num_cores=NC, num_subcores=NS)
    out_type = [jax.ShapeDtypeStruct((N_PAD, width), jnp.float32)] * 2
    if with_deg:
        out_type += [jax.ShapeDtypeStruct((N_PAD, 2), jnp.float32)] * 2

    scratch = [
        pltpu.VMEM((CPT, CH), jnp.int32),
        pltpu.VMEM((CPT, CH), jnp.int32),
        pltpu.VMEM((CH, width), jnp.float32),
        pltpu.VMEM_SHARED((N_PAD, width), jnp.float32),
        pltpu.VMEM_SHARED((N_PAD, width), jnp.float32),
        pltpu.SemaphoreType.DMA,
    ]
    if with_deg:
        scratch += [
            pltpu.VMEM((CH, 2), jnp.float32),
            pltpu.VMEM((RPT, 2), jnp.float32),
            pltpu.VMEM_SHARED((N_PAD, 2), jnp.float32),
        ]

    def body(y_hbm, edges_hbm, *rest):
        if with_deg:
            (ones_hbm, zeros_hbm, out0_hbm, out1_hbm, deg0_hbm, deg1_hbm,
             src_v, dst_v, rows_v, acc, y_s, sem,
             ones_v, dz_v, dega) = rest
        else:
            (out0_hbm, out1_hbm, src_v, dst_v, rows_v, acc, y_s, sem) = rest

        c = lax.axis_index("c")
        s = lax.axis_index("s")
        wid = s * NC + c

        zero16 = jnp.zeros((16,), jnp.float32)

        def zrow(i, _):
            for j in range(width // 16):
                rows_v[i, pl.ds(j * 16, 16)] = zero16
            return 0

        lax.fori_loop(0, RB, zrow, 0)
        rv_rb = rows_v.at[pl.ds(0, RB)]
        for t in range(RPT // RB):
            r0 = s * RPT + t * RB
            pltpu.sync_copy(rv_rb, acc.at[pl.ds(r0, RB)])

        if with_deg:
            pltpu.sync_copy(ones_hbm, ones_v)
            pltpu.sync_copy(zeros_hbm.at[pl.ds(s * RPT, RPT)], dz_v)
            pltpu.sync_copy(dz_v, dega.at[pl.ds(s * RPT, RPT)])

        pltpu.sync_copy(edges_hbm.at[0, wid], src_v)
        pltpu.sync_copy(edges_hbm.at[1, wid], dst_v)
        for t in range(RPT // RB):
            r0 = s * RPT + t * RB
            pltpu.sync_copy(y_hbm.at[pl.ds(r0, RB)], rv_rb)
            pltpu.sync_copy(rv_rb, y_s.at[pl.ds(r0, RB)])

        plsc.subcore_barrier()

        def step(j, _):
            pltpu.async_copy(y_s.at[src_v.at[j]], rows_v, sem).wait()
            pltpu.sync_copy(rows_v, acc.at[dst_v.at[j]], add=True)
            if with_deg:
                pltpu.sync_copy(ones_v, dega.at[dst_v.at[j]], add=True)
            return 0

        lax.fori_loop(0, CPT, step, 0)

        plsc.subcore_barrier()

        def writeout(out_hbm, deg_hbm):
            for t in range(RPT // RB):
                r0 = s * RPT + t * RB
                pltpu.sync_copy(acc.at[pl.ds(r0, RB)], rv_rb)
                pltpu.sync_copy(rv_rb, out_hbm.at[pl.ds(r0, RB)])
            if with_deg:
                pltpu.sync_copy(dega.at[pl.ds(s * RPT, RPT)], dz_v)
                pltpu.sync_copy(dz_v, deg_hbm.at[pl.ds(s * RPT, RPT)])

        @pl.when(c == 0)
        def _():
            writeout(out0_hbm, deg0_hbm if with_deg else None)

        @pl.when(c == 1)
        def _():
            writeout(out1_hbm, deg1_hbm if with_deg else None)

    return pl.kernel(body, out_type=out_type, mesh=mesh,
                     scratch_types=scratch,
                     compiler_params=pltpu.CompilerParams(
                         use_tc_tiling_on_sc=False))


def _mm_body(x_ref, w_ref, y_ref, r_ref):
    y_ref[...] = jnp.dot(x_ref[...], w_ref[:, :HID],
                         preferred_element_type=jnp.float32)
    r_ref[...] = jnp.dot(x_ref[...], w_ref[:, HID:],
                         preferred_element_type=jnp.float32)


def _combine_body(p0, p1, d0, d1, r1, b, w, y_ref, r_ref):
    rdeg = 1.0 / jnp.maximum(d0[:, :1] + d1[:, :1], 1.0)
    h = jnp.maximum((p0[...] + p1[...]) * rdeg + b[...] + r1[...], 0.0)
    y_ref[...] = jnp.dot(h, w[:, :HID], preferred_element_type=jnp.float32)
    r_ref[...] = jnp.dot(h, w[:, HID:], preferred_element_type=jnp.float32)


def _final_body(q0, q1, d0, d1, r2, b, wc, bc, o_ref):
    rdeg = 1.0 / jnp.maximum(d0[:, :1] + d1[:, :1], 1.0)
    h = jnp.maximum((q0[...] + q1[...]) * rdeg + b[...] + r2[...], 0.0)
    o_ref[...] = jnp.sum(h * wc[...], axis=1) + bc[0, 0]


def _full(shape):
    nd = len(shape)
    return pl.BlockSpec(shape, lambda i: (0,) * nd)


_row64 = pl.BlockSpec((R, HID), lambda i: (i, 0))
_row2 = pl.BlockSpec((R, 2), lambda i: (i, 0))
_rowf = pl.BlockSpec((R,), lambda i: (i,))
_o64 = jax.ShapeDtypeStruct((N_PAD, HID), jnp.float32)


def _mm(x, w):
    return pl.pallas_call(
        _mm_body,
        grid=(GRID,),
        in_specs=[pl.BlockSpec((R, D_IN), lambda i: (i, 0)),
                  _full((D_IN, 2 * HID))],
        out_specs=[_row64, _row64],
        out_shape=[_o64, _o64],
    )(x, w)


def _combine(p0, p1, d0, d1, r1, b, w):
    return pl.pallas_call(
        _combine_body,
        grid=(GRID,),
        in_specs=[_row64, _row64, _row2, _row2, _row64,
                  _full((1, HID)), _full((HID, 2 * HID))],
        out_specs=[_row64, _row64],
        out_shape=[_o64, _o64],
    )(p0, p1, d0, d1, r1, b, w)


def _final(q0, q1, d0, d1, r2, b, wc, bc):
    return pl.pallas_call(
        _final_body,
        grid=(GRID,),
        in_specs=[_row64, _row64, _row2, _row2, _row64,
                  _full((1, HID)), _full((1, HID)), _full((1, 1))],
        out_specs=_rowf,
        out_shape=jax.ShapeDtypeStruct((N_PAD,), jnp.float32),
    )(q0, q1, d0, d1, r2, b, wc, bc)


_agg_cache = {}


def _get_agg(with_deg):
    if with_deg not in _agg_cache:
        _agg_cache[with_deg] = _make_sc_agg(with_deg, HID)
    return _agg_cache[with_deg]


@jax.jit
def kernel(x, edge_index, W1l, b1, W1r, W2l, b2, W2r, Wc, bc):
    ei = edge_index.astype(jnp.int32)
    edges = jnp.pad(ei, ((0, 0), (0, E_PAD - E)),
                    constant_values=N).reshape(2, NW, CPT, CH)

    x_pad = jnp.zeros((N_PAD, D_IN), jnp.float32).at[:N].set(x)
    wcat1 = jnp.concatenate([W1l.T, W1r.T], axis=1)
    wcat2 = jnp.concatenate([W2l.T, W2r.T], axis=1)
    ones2 = jnp.ones((CH, 2), jnp.float32)
    zeros2 = jnp.zeros((N_PAD, 2), jnp.float32)

    y1, r1 = _mm(x_pad, wcat1)
    p0, p1, d0, d1 = _get_agg(True)(y1, edges, ones2, zeros2)
    y2, r2 = _combine(p0, p1, d0, d1, r1, b1.reshape(1, HID), wcat2)

    q0, q1 = _get_agg(False)(y2, edges)
    logits = _final(q0, q1, d0, d1, r2, b2.reshape(1, HID),
                    Wc.reshape(1, HID), bc.reshape(1, 1))
    return logits[:N]


# speedup vs baseline: 2.2695x; 1.0225x over previous
import jax
import jax.numpy as jnp
from jax import lax
from jax.experimental import pallas as pl
from jax.experimental.pallas import tpu as pltpu
from jax.experimental.pallas import tpu_sc as plsc

N = 10000
E = 320000
D_IN = 128
HID = 64

NC = 2
NS = 16
NW = NC * NS
CH = 256
CPT = -(-E // (NW * CH))
E_PAD = NW * CPT * CH
N_PAD = 10240
RPT = N_PAD // NS
RB = 128
R = 1024
GRID = N_PAD // R


def _make_sc_agg(with_deg: bool, width: int):
    mesh = plsc.VectorSubcoreMesh(core_axis_name="c", subcore_axis_name="s",
                                  num_cores=NC, num_subcores=NS)
    out_type = [jax.ShapeDtypeStruct((N_PAD, width), jnp.float32)] * 2
    if with_deg:
        out_type += [jax.ShapeDtypeStruct((N_PAD,), jnp.float32)] * 2

    scratch = [
        pltpu.VMEM((CPT, CH), jnp.int32),
        pltpu.VMEM((CPT, CH), jnp.int32),
        pltpu.VMEM((CH, width), jnp.float32),
        pltpu.VMEM_SHARED((N_PAD, width), jnp.float32),
        pltpu.VMEM_SHARED((N_PAD, width), jnp.float32),
        pltpu.SemaphoreType.DMA,
    ]
    if with_deg:
        scratch += [
            pltpu.VMEM((CH,), jnp.float32),
            pltpu.VMEM((RPT,), jnp.float32),
            pltpu.VMEM_SHARED((N_PAD,), jnp.float32),
        ]

    def body(y_hbm, edges_hbm, *rest):
        if with_deg:
            (out0_hbm, out1_hbm, deg0_hbm, deg1_hbm,
             src_v, dst_v, rows_v, acc, y_s, sem,
             ones_v, dz_v, dega) = rest
        else:
            (out0_hbm, out1_hbm, src_v, dst_v, rows_v, acc, y_s, sem) = rest

        c = lax.axis_index("c")
        s = lax.axis_index("s")
        wid = s * NC + c

        zero16 = jnp.zeros((16,), jnp.float32)

        def zrow(i, _):
            for j in range(width // 16):
                rows_v[i, pl.ds(j * 16, 16)] = zero16
            return 0

        lax.fori_loop(0, RB, zrow, 0)
        rv_rb = rows_v.at[pl.ds(0, RB)]
        for t in range(RPT // RB):
            r0 = s * RPT + t * RB
            pltpu.sync_copy(rv_rb, acc.at[pl.ds(r0, RB)])

        if with_deg:
            for j in range(CH // 16):
                ones_v[pl.ds(j * 16, 16)] = jnp.ones((16,), jnp.float32)
            for j in range(RPT // 16):
                dz_v[pl.ds(j * 16, 16)] = zero16
            pltpu.sync_copy(dz_v, dega.at[pl.ds(s * RPT, RPT)])

        pltpu.sync_copy(edges_hbm.at[0, wid], src_v)
        pltpu.sync_copy(edges_hbm.at[1, wid], dst_v)
        for t in range(RPT // RB):
            r0 = s * RPT + t * RB
            pltpu.sync_copy(y_hbm.at[pl.ds(r0, RB)], rv_rb)
            pltpu.sync_copy(rv_rb, y_s.at[pl.ds(r0, RB)])

        plsc.subcore_barrier()

        def step(j, _):
            pltpu.async_copy(y_s.at[src_v.at[j]], rows_v, sem).wait()
            pltpu.sync_copy(rows_v, acc.at[dst_v.at[j]], add=True)
            if with_deg:
                pltpu.sync_copy(ones_v, dega.at[dst_v.at[j]], add=True)
            return 0

        lax.fori_loop(0, CPT, step, 0)

        plsc.subcore_barrier()

        def writeout(out_hbm, deg_hbm):
            for t in range(RPT // RB):
                r0 = s * RPT + t * RB
                pltpu.sync_copy(acc.at[pl.ds(r0, RB)], rv_rb)
                pltpu.sync_copy(rv_rb, out_hbm.at[pl.ds(r0, RB)])
            if with_deg:
                pltpu.sync_copy(dega.at[pl.ds(s * RPT, RPT)], dz_v)
                pltpu.sync_copy(dz_v, deg_hbm.at[pl.ds(s * RPT, RPT)])

        @pl.when(c == 0)
        def _():
            writeout(out0_hbm, deg0_hbm if with_deg else None)

        @pl.when(c == 1)
        def _():
            writeout(out1_hbm, deg1_hbm if with_deg else None)

    return pl.kernel(body, out_type=out_type, mesh=mesh,
                     scratch_types=scratch,
                     compiler_params=pltpu.CompilerParams(
                         use_tc_tiling_on_sc=False))


def _mm_body(x_ref, w_ref, y_ref, r_ref):
    y_ref[...] = jnp.dot(x_ref[...], w_ref[:, :HID],
                         preferred_element_type=jnp.float32)
    r_ref[...] = jnp.dot(x_ref[...], w_ref[:, HID:],
                         preferred_element_type=jnp.float32)


def _combine_body(p0, p1, d0, d1, r1, b, w, y_ref, r_ref):
    rdeg = 1.0 / jnp.maximum(d0[...] + d1[...], 1.0)
    h = jnp.maximum((p0[...] + p1[...]) * rdeg + b[...] + r1[...], 0.0)
    y_ref[...] = jnp.dot(h, w[:, :HID], preferred_element_type=jnp.float32)
    r_ref[...] = jnp.dot(h, w[:, HID:], preferred_element_type=jnp.float32)


def _final_body(q0, q1, d0, d1, r2, b, wc, bc, o_ref):
    rdeg = 1.0 / jnp.maximum(d0[...] + d1[...], 1.0)
    h = jnp.maximum((q0[...] + q1[...]) * rdeg + b[...] + r2[...], 0.0)
    o_ref[...] = jnp.sum(h * wc[...], axis=1) + bc[0, 0]


def _full(shape):
    nd = len(shape)
    return pl.BlockSpec(shape, lambda i: (0,) * nd)


_row64 = pl.BlockSpec((R, HID), lambda i: (i, 0))
_row2 = pl.BlockSpec((R, 1), lambda i: (i, 0))
_rowf = pl.BlockSpec((R,), lambda i: (i,))
_o64 = jax.ShapeDtypeStruct((N_PAD, HID), jnp.float32)


def _mm(x, w):
    return pl.pallas_call(
        _mm_body,
        grid=(GRID,),
        in_specs=[pl.BlockSpec((R, D_IN), lambda i: (i, 0)),
                  _full((D_IN, 2 * HID))],
        out_specs=[_row64, _row64],
        out_shape=[_o64, _o64],
    )(x, w)


def _combine(p0, p1, d0, d1, r1, b, w):
    return pl.pallas_call(
        _combine_body,
        grid=(GRID,),
        in_specs=[_row64, _row64, _row2, _row2, _row64,
                  _full((1, HID)), _full((HID, 2 * HID))],
        out_specs=[_row64, _row64],
        out_shape=[_o64, _o64],
    )(p0, p1, d0, d1, r1, b, w)


def _final(q0, q1, d0, d1, r2, b, wc, bc):
    return pl.pallas_call(
        _final_body,
        grid=(GRID,),
        in_specs=[_row64, _row64, _row2, _row2, _row64,
                  _full((1, HID)), _full((1, HID)), _full((1, 1))],
        out_specs=_rowf,
        out_shape=jax.ShapeDtypeStruct((N_PAD,), jnp.float32),
    )(q0, q1, d0, d1, r2, b, wc, bc)


_agg_cache = {}


def _get_agg(with_deg):
    if with_deg not in _agg_cache:
        _agg_cache[with_deg] = _make_sc_agg(with_deg, HID)
    return _agg_cache[with_deg]


@jax.jit
def kernel(x, edge_index, W1l, b1, W1r, W2l, b2, W2r, Wc, bc):
    ei = edge_index.astype(jnp.int32)
    edges = jnp.pad(ei, ((0, 0), (0, E_PAD - E)),
                    constant_values=N).reshape(2, NW, CPT, CH)

    x_pad = jnp.zeros((N_PAD, D_IN), jnp.float32).at[:N].set(x)
    wcat1 = jnp.concatenate([W1l.T, W1r.T], axis=1)
    wcat2 = jnp.concatenate([W2l.T, W2r.T], axis=1)
    y1, r1 = _mm(x_pad, wcat1)
    p0, p1, dg0, dg1 = _get_agg(True)(y1, edges)
    d0 = dg0.reshape(N_PAD, 1)
    d1 = dg1.reshape(N_PAD, 1)
    y2, r2 = _combine(p0, p1, d0, d1, r1, b1.reshape(1, HID), wcat2)

    q0, q1 = _get_agg(False)(y2, edges)
    logits = _final(q0, q1, d0, d1, r2, b2.reshape(1, HID),
                    Wc.reshape(1, HID), bc.reshape(1, 1))
    return logits[:N]
